# Initial kernel scaffold; baseline (speedup 1.0000x reference)
import jax
import jax.numpy as jnp
from jax.experimental import pallas as pl


def kernel(x_v, x_e, edge_orders, b_table):
    raise NotImplementedError("write your pallas kernel here")


# speedup vs baseline: 1.4730x; 1.4730x over previous
import functools

import jax
import jax.numpy as jnp
from jax import lax
from jax.experimental import pallas as pl
from jax.experimental.pallas import tpu as pltpu
from jax.experimental.pallas import tpu_sc as plsc

_DIM = 128
_NROWS = 11
_NC, _NS = 2, 16
_NW = _NC * _NS
_BLK = 128
_LANES = 16


def _make_xe_kernel(n_edges):
    nblk = n_edges // _BLK
    jmax = -(-nblk // _NW)

    mesh = plsc.VectorSubcoreMesh(
        core_axis_name="c", subcore_axis_name="s",
        num_cores=_NC, num_subcores=_NS,
    )

    @functools.partial(
        pl.kernel,
        out_type=jax.ShapeDtypeStruct((n_edges, _DIM), jnp.float32),
        mesh=mesh,
        scratch_types=[
            pltpu.VMEM((_NROWS, _DIM), jnp.float32),
            pltpu.VMEM((_BLK,), jnp.int32),
            pltpu.VMEM((_BLK, _DIM), jnp.float32),
        ],
    )
    def xe_kernel(x_e, orders, btab, out, btab_v, idx_v, buf):
        wid = lax.axis_index("s") * _NC + lax.axis_index("c")
        pltpu.sync_copy(btab, btab_v)

        @pl.loop(0, jmax)
        def _(j):
            bid = wid + _NW * j

            @pl.when(bid < nblk)
            def _():
                base = bid * _BLK
                pltpu.sync_copy(orders.at[pl.ds(base, _BLK)], idx_v)
                pltpu.sync_copy(x_e.at[pl.ds(base, _BLK)], buf)

                @pl.loop(0, _BLK // _LANES)
                def _(g):
                    ovec = idx_v[pl.ds(g * _LANES, _LANES)]
                    for r in range(_LANES):
                        o = ovec[r]
                        row = g * _LANES + r
                        for v in range(_DIM // _LANES):
                            sl = pl.ds(v * _LANES, _LANES)
                            plsc.addupdate(buf.at[row, sl], btab_v[o, sl])

                pltpu.sync_copy(buf, out.at[pl.ds(base, _BLK)])

    return xe_kernel


def _xv_body(xv_ref, b_ref, out_ref):
    out_ref[...] = xv_ref[...] + b_ref[1:2, :]


def _xv_add(x_v, b_table):
    n = x_v.shape[0]
    blk = 2000
    return pl.pallas_call(
        _xv_body,
        out_shape=jax.ShapeDtypeStruct((n, _DIM), jnp.float32),
        in_specs=[
            pl.BlockSpec((blk, _DIM), lambda i: (i, 0)),
            pl.BlockSpec((_NROWS, _DIM), lambda i: (0, 0)),
        ],
        out_specs=pl.BlockSpec((blk, _DIM), lambda i: (i, 0)),
        grid=(n // blk,),
    )(x_v, b_table)


def kernel(x_v, x_e, edge_orders, b_table):
    xe_out = _make_xe_kernel(x_e.shape[0])(x_e, edge_orders, b_table)
    xv_out = _xv_add(x_v, b_table)
    return (xv_out, xe_out)
